# baseline (speedup 1.0000x reference)
import functools

import numpy as np
import jax
import jax.numpy as jnp
from jax import lax
from jax.experimental import pallas as pl
from jax.experimental.pallas import tpu as pltpu
from jax.experimental.pallas import tpu_sc as plsc

D_M = 128
BATCH = 1024
MAXLEN = 200
N_TOK = BATCH * MAXLEN
NC, NS, LANES = 2, 16, 16
NW = NC * NS
PER_W = N_TOK // NW
CHUNK = 80
NCHUNK = PER_W // CHUNK
PE_EXT = MAXLEN + CHUNK


def _pe_table() -> jnp.ndarray:
    pos = (np.arange(PE_EXT, dtype=np.int64) % MAXLEN).astype(np.float64)[:, None]
    i = np.arange(D_M)[None, :]
    angle = pos / np.power(10000.0, (2.0 * (i // 2)) / float(D_M))
    pe = np.where(i % 2 == 0, np.sin(angle), np.cos(angle))
    return jnp.asarray(pe, dtype=jnp.float32)


NBUF = 8
LP = 6
LG = 3

_mesh = plsc.VectorSubcoreMesh(core_axis_name="c", subcore_axis_name="s")


@functools.partial(
    pl.kernel,
    out_type=jax.ShapeDtypeStruct((N_TOK, D_M), jnp.float32),
    mesh=_mesh,
    scratch_types=[
        pltpu.VMEM((PER_W,), jnp.int32),
        pltpu.VMEM_SHARED((PE_EXT, D_M), jnp.float32),
        pltpu.VMEM((NBUF, CHUNK, D_M), jnp.float32),
    ] + [pltpu.SemaphoreType.DMA] * (3 * NBUF + 2),
)
def _emb_kernel(idx_hbm, pe_hbm, table_hbm, out_hbm, idx_v, pe_v, rows_v,
                *sems):
    sem_g = sems[:NBUF]
    sem_o = sems[NBUF:2 * NBUF]
    sem_p = sems[2 * NBUF:3 * NBUF]
    sem_pe, sem_i = sems[3 * NBUF:]
    sid = lax.axis_index("s")
    wid = sid * NC + lax.axis_index("c")
    base = wid * PER_W

    @pl.when(sid == 0)
    def _():
        pltpu.async_copy(pe_hbm, pe_v, sem_pe)

    pltpu.async_copy(idx_hbm.at[pl.ds(base, PER_W)], idx_v, sem_i)

    def start_prefill(t, b):
        prow = lax.rem(t * CHUNK, MAXLEN)
        pltpu.async_copy(pe_v.at[pl.ds(prow, CHUNK)], rows_v.at[b], sem_p[b])

    def start_gather(t, b):
        pltpu.make_async_copy(
            pe_v.at[pl.ds(0, CHUNK)], rows_v.at[b], sem_p[b]).wait()
        pltpu.async_copy(
            table_hbm.at[idx_v.at[pl.ds(t * CHUNK, CHUNK)]],
            rows_v.at[b], sem_g[b], add=True)

    def wait_gather(b):
        pltpu.make_async_copy(
            table_hbm.at[idx_v.at[pl.ds(0, CHUNK)]],
            rows_v.at[b], sem_g[b]).wait()

    def start_out(t, b):
        pltpu.async_copy(
            rows_v.at[b], out_hbm.at[pl.ds(base + t * CHUNK, CHUNK)],
            sem_o[b])

    def wait_out(b):
        pltpu.make_async_copy(
            rows_v.at[b], out_hbm.at[pl.ds(base, CHUNK)], sem_o[b]).wait()

    for b in range(LP):
        pltpu.async_copy(
            pe_hbm.at[pl.ds((b * CHUNK) % MAXLEN, CHUNK)], rows_v.at[b],
            sem_p[b])
    pltpu.make_async_copy(
        idx_hbm.at[pl.ds(base, PER_W)], idx_v, sem_i).wait()
    for b in range(LG):
        start_gather(b, b)

    @pl.when(sid == 0)
    def _():
        pltpu.make_async_copy(pe_hbm, pe_v, sem_pe).wait()

    plsc.subcore_barrier()

    NGROUP = NCHUNK // NBUF

    def staged(g, lo, hi, fn):
        if lo <= 0 and hi >= NGROUP - 1:
            fn()
        elif lo <= hi:
            cond = (g >= lo) if hi >= NGROUP - 1 else (
                (g <= hi) if lo <= 0 else (g >= lo) & (g <= hi))

            @pl.when(cond)
            def _():
                fn()

    def group_body(g, carry):
        for b in range(NBUF):
            u = g * NBUF + b
            a0 = b + LP
            ba = a0 % NBUF
            lo_w = 1 if a0 < NBUF else 0
            staged(g, lo_w, (NCHUNK - 1 - a0) // NBUF, lambda: wait_out(ba))
            staged(g, 0, (NCHUNK - 1 - a0) // NBUF,
                   lambda: start_prefill(g * NBUF + a0, ba))
            v0 = b + LG
            bv = v0 % NBUF
            staged(g, 0, (NCHUNK - 1 - v0) // NBUF,
                   lambda: start_gather(g * NBUF + v0, bv))
            wait_gather(b)
            start_out(u, b)
        return carry

    lax.fori_loop(0, NGROUP, group_body, 0)
    for b in range(NBUF):
        wait_out(b)


def kernel(inp, emb_table):
    idx = inp.reshape(N_TOK).astype(jnp.int32)
    out = _emb_kernel(idx, _pe_table(), emb_table)
    return out.reshape(BATCH, MAXLEN, D_M)


# speedup vs baseline: 1.0624x; 1.0624x over previous
import functools

import numpy as np
import jax
import jax.numpy as jnp
from jax import lax
from jax.experimental import pallas as pl
from jax.experimental.pallas import tpu as pltpu
from jax.experimental.pallas import tpu_sc as plsc

D_M = 128
BATCH = 1024
MAXLEN = 200
N_TOK = BATCH * MAXLEN
NC, NS, LANES = 2, 16, 16
NW = NC * NS
PER_W = N_TOK // NW
CHUNK = 40
NCHUNK = PER_W // CHUNK
PE_EXT = MAXLEN + CHUNK


def _pe_table() -> jnp.ndarray:
    pos = (np.arange(PE_EXT, dtype=np.int64) % MAXLEN).astype(np.float64)[:, None]
    i = np.arange(D_M)[None, :]
    angle = pos / np.power(10000.0, (2.0 * (i // 2)) / float(D_M))
    pe = np.where(i % 2 == 0, np.sin(angle), np.cos(angle))
    return jnp.asarray(pe, dtype=jnp.float32)


NBUF = 10
LP = 7
LG = 4

_mesh = plsc.VectorSubcoreMesh(core_axis_name="c", subcore_axis_name="s")


@functools.partial(
    pl.kernel,
    out_type=jax.ShapeDtypeStruct((N_TOK, D_M), jnp.float32),
    mesh=_mesh,
    scratch_types=[
        pltpu.VMEM((PER_W,), jnp.int32),
        pltpu.VMEM_SHARED((PE_EXT, D_M), jnp.float32),
        pltpu.VMEM((NBUF, CHUNK, D_M), jnp.float32),
    ] + [pltpu.SemaphoreType.DMA] * (3 * NBUF),
)
def _emb_kernel(idx_hbm, pe_hbm, table_hbm, out_hbm, idx_v, pe_v, rows_v,
                *sems):
    sem_g = sems[:NBUF]
    sem_o = sems[NBUF:2 * NBUF]
    sem_p = sems[2 * NBUF:3 * NBUF]
    sid = lax.axis_index("s")
    wid = sid * NC + lax.axis_index("c")
    base = wid * PER_W

    @pl.when(sid == 0)
    def _():
        pltpu.sync_copy(pe_hbm, pe_v)

    pltpu.sync_copy(idx_hbm.at[pl.ds(base, PER_W)], idx_v)
    plsc.subcore_barrier()

    def start_prefill(t, b):
        prow = lax.rem(t * CHUNK, MAXLEN)
        pltpu.async_copy(pe_v.at[pl.ds(prow, CHUNK)], rows_v.at[b], sem_p[b])

    def start_gather(t, b):
        pltpu.make_async_copy(
            pe_v.at[pl.ds(0, CHUNK)], rows_v.at[b], sem_p[b]).wait()
        pltpu.async_copy(
            table_hbm.at[idx_v.at[pl.ds(t * CHUNK, CHUNK)]],
            rows_v.at[b], sem_g[b], add=True)

    def wait_gather(b):
        pltpu.make_async_copy(
            table_hbm.at[idx_v.at[pl.ds(0, CHUNK)]],
            rows_v.at[b], sem_g[b]).wait()

    def start_out(t, b):
        pltpu.async_copy(
            rows_v.at[b], out_hbm.at[pl.ds(base + t * CHUNK, CHUNK)],
            sem_o[b])

    def wait_out(b):
        pltpu.make_async_copy(
            rows_v.at[b], out_hbm.at[pl.ds(base, CHUNK)], sem_o[b]).wait()

    for b in range(LP):
        start_prefill(b, b)
    for b in range(LG):
        start_gather(b, b)

    NGROUP = NCHUNK // NBUF

    def staged(g, lo, hi, fn):
        if lo <= 0 and hi >= NGROUP - 1:
            fn()
        elif lo <= hi:
            cond = (g >= lo) if hi >= NGROUP - 1 else (
                (g <= hi) if lo <= 0 else (g >= lo) & (g <= hi))

            @pl.when(cond)
            def _():
                fn()

    def group_body(g, carry):
        for b in range(NBUF):
            u = g * NBUF + b
            a0 = b + LP
            ba = a0 % NBUF
            lo_w = 1 if a0 < NBUF else 0
            staged(g, lo_w, (NCHUNK - 1 - a0) // NBUF, lambda: wait_out(ba))
            staged(g, 0, (NCHUNK - 1 - a0) // NBUF,
                   lambda: start_prefill(g * NBUF + a0, ba))
            v0 = b + LG
            bv = v0 % NBUF
            staged(g, 0, (NCHUNK - 1 - v0) // NBUF,
                   lambda: start_gather(g * NBUF + v0, bv))
            wait_gather(b)
            start_out(u, b)
        return carry

    lax.fori_loop(0, NGROUP, group_body, 0)
    for b in range(NBUF):
        wait_out(b)


def kernel(inp, emb_table):
    idx = inp.reshape(N_TOK).astype(jnp.int32)
    out = _emb_kernel(idx, _pe_table(), emb_table)
    return out.reshape(BATCH, MAXLEN, D_M)
